# baseline (speedup 1.0000x reference)
import jax
import jax.numpy as jnp
from jax import lax
from jax.experimental import pallas as pl
from jax.experimental.pallas import tpu as pltpu
from jax.experimental.pallas import tpu_sc as plsc

NUM_NODES = 1000000
NUM_RELATIONS = 1000
HIDDEN = 64
BATCH = 16384

NUM_CORES = 2
NUM_SUBCORES = 16
LANES = 16
NW = NUM_CORES * NUM_SUBCORES
BPW = BATCH // NW
CHUNK = 128
NCHUNK = BPW // CHUNK
VPC = CHUNK // LANES


def _newton_rsqrt(x):
    i = plsc.bitcast(x, jnp.int32)
    i = jnp.int32(0x5F3759DF) - (i >> 1)
    y = plsc.bitcast(i, jnp.float32)
    for _ in range(3):
        y = y * (1.5 - 0.5 * x * y * y)
    return y


def _body(head_hbm, rel_hbm, tail_hbm, ent_hbm, relemb_hbm, out_hbm,
          idx_h, idx_r, idx_t, gidx_h, gidx_r, gidx_t,
          feat_h, feat_r, feat_t, out_v, sem):
    wid = lax.axis_index("s") * NUM_CORES + lax.axis_index("c")
    base = wid * BPW

    for j in range(NCHUNK):
        pltpu.sync_copy(head_hbm.at[pl.ds(base + j * CHUNK, CHUNK)], idx_h.at[j])
        pltpu.sync_copy(rel_hbm.at[pl.ds(base + j * CHUNK, CHUNK)], idx_r.at[j])
        pltpu.sync_copy(tail_hbm.at[pl.ds(base + j * CHUNK, CHUNK)], idx_t.at[j])

    def build_and_start(j):
        s = j % 2

        def fill(c, carry):
            for v in range(VPC):
                sl = pl.ds(v * LANES, LANES)
                dsl = pl.dslice(c * CHUNK + v * LANES, LANES)
                gidx_h[s, dsl] = idx_h[j, sl] + c * NUM_NODES
                gidx_t[s, dsl] = idx_t[j, sl] + c * NUM_NODES
                gidx_r[s, dsl] = idx_r[j, sl] + c * NUM_RELATIONS
            return carry

        lax.fori_loop(0, HIDDEN, fill, None)
        return (pltpu.async_copy(ent_hbm.at[gidx_h.at[s]], feat_h.at[s], sem),
                pltpu.async_copy(relemb_hbm.at[gidx_r.at[s]], feat_r.at[s], sem),
                pltpu.async_copy(ent_hbm.at[gidx_t.at[s]], feat_t.at[s], sem))

    lane = lax.iota(jnp.int32, LANES)
    zero = jnp.zeros((LANES,), jnp.float32)

    def make_group(j):
        s = j % 2

        def group(g, carry):
            hh = zero; tt = zero; rr = zero
            hr = zero; ht = zero; rt = zero
            for c in range(HIDDEN):
                csl = pl.ds(c * CHUNK + g * LANES, LANES)
                h = feat_h[s, csl]
                r = feat_r[s, csl]
                t = feat_t[s, csl]
                hh = hh + h * h
                tt = tt + t * t
                rr = rr + r * r
                hr = hr + h * r
                ht = ht + h * t
                rt = rt + r * t
            inv_h = _newton_rsqrt(jnp.maximum(hh, 1e-24))
            inv_t = _newton_rsqrt(jnp.maximum(tt, 1e-24))
            d2 = (rr + hh * inv_h * inv_h + tt * inv_t * inv_t
                  + 2.0 * (hr * inv_h - ht * (inv_h * inv_t) - rt * inv_t))
            d2 = jnp.maximum(d2, 0.0)
            out_v[pl.ds(j * CHUNK + g * LANES, LANES)] = (
                d2 * _newton_rsqrt(jnp.maximum(d2, 1e-24)))
            return carry

        return group

    pending = build_and_start(0)
    for j in range(NCHUNK):
        for c in pending:
            c.wait()
        nxt = build_and_start(j + 1) if j + 1 < NCHUNK else None
        lax.fori_loop(0, VPC, make_group(j), None)
        pending = nxt

    pltpu.sync_copy(out_v, out_hbm.at[pl.ds(base, BPW)])


def _transe_sc(head_index, rel_index, tail_index, ent_flat, rel_flat):
    mesh = plsc.VectorSubcoreMesh(core_axis_name="c", subcore_axis_name="s")
    f = pl.kernel(
        _body,
        out_type=jax.ShapeDtypeStruct((BATCH,), jnp.float32),
        mesh=mesh,
        scratch_types=[
            pltpu.VMEM((NCHUNK, CHUNK), jnp.int32),
            pltpu.VMEM((NCHUNK, CHUNK), jnp.int32),
            pltpu.VMEM((NCHUNK, CHUNK), jnp.int32),
            pltpu.VMEM((2, HIDDEN * CHUNK), jnp.int32),
            pltpu.VMEM((2, HIDDEN * CHUNK), jnp.int32),
            pltpu.VMEM((2, HIDDEN * CHUNK), jnp.int32),
            pltpu.VMEM((2, HIDDEN * CHUNK), jnp.float32),
            pltpu.VMEM((2, HIDDEN * CHUNK), jnp.float32),
            pltpu.VMEM((2, HIDDEN * CHUNK), jnp.float32),
            pltpu.VMEM((BPW,), jnp.float32),
            pltpu.SemaphoreType.DMA,
        ],
        compiler_params=pltpu.CompilerParams(
            needs_layout_passes=False, use_tc_tiling_on_sc=False),
        name="transe_sc",
    )
    return f(head_index, rel_index, tail_index, ent_flat, rel_flat)


def kernel(head_index, rel_index, tail_index, ent_emb, rel_emb):
    ent_flat = jnp.reshape(jnp.transpose(ent_emb), (NUM_NODES * HIDDEN,))
    rel_flat = jnp.reshape(jnp.transpose(rel_emb), (NUM_RELATIONS * HIDDEN,))
    return _transe_sc(head_index.astype(jnp.int32), rel_index.astype(jnp.int32),
                      tail_index.astype(jnp.int32), ent_flat, rel_flat)
